# Initial kernel scaffold; baseline (speedup 1.0000x reference)
import jax
import jax.numpy as jnp
from jax.experimental import pallas as pl


def kernel(pos, seg_embd_weight):
    raise NotImplementedError("write your pallas kernel here")


# speedup vs baseline: 23.0065x; 23.0065x over previous
import jax
import jax.numpy as jnp
from jax.experimental import pallas as pl

_ROWS = 16384
_SEQ = 200
_DIM = 128
_BLK = 64


def _embed_kernel(pos_ref, w_ref, out_ref):
    posf = pos_ref[...].astype(jnp.float32)
    w0 = w_ref[0, :]
    diff = w_ref[1, :] - w0
    out_ref[...] = posf[:, :, None] * diff[None, None, :] + w0[None, None, :]


def kernel(pos, seg_embd_weight):
    pos = pos.astype(jnp.int32)
    grid = (_ROWS // _BLK,)
    return pl.pallas_call(
        _embed_kernel,
        grid=grid,
        in_specs=[
            pl.BlockSpec((_BLK, _SEQ), lambda i: (i, 0)),
            pl.BlockSpec((2, _DIM), lambda i: (0, 0)),
        ],
        out_specs=pl.BlockSpec((_BLK, _SEQ, _DIM), lambda i: (i, 0, 0)),
        out_shape=jax.ShapeDtypeStruct((_ROWS, _SEQ, _DIM), jnp.float32),
    )(pos, seg_embd_weight)


# speedup vs baseline: 24.9951x; 1.0864x over previous
import jax
import jax.numpy as jnp
from jax.experimental import pallas as pl

_ROWS = 16384
_SEQ = 200
_DIM = 128
_BLK = 128


def _embed_kernel(pos_ref, w_ref, out_ref):
    posf = pos_ref[...].astype(jnp.float32)
    w0 = w_ref[0, :]
    diff = w_ref[1, :] - w0
    out_ref[...] = posf[:, :, None] * diff[None, None, :] + w0[None, None, :]


def kernel(pos, seg_embd_weight):
    pos = pos.astype(jnp.int32)
    grid = (_ROWS // _BLK,)
    return pl.pallas_call(
        _embed_kernel,
        grid=grid,
        in_specs=[
            pl.BlockSpec((_BLK, _SEQ), lambda i: (i, 0)),
            pl.BlockSpec((2, _DIM), lambda i: (0, 0)),
        ],
        out_specs=pl.BlockSpec((_BLK, _SEQ, _DIM), lambda i: (i, 0, 0)),
        out_shape=jax.ShapeDtypeStruct((_ROWS, _SEQ, _DIM), jnp.float32),
    )(pos, seg_embd_weight)


# speedup vs baseline: 25.1659x; 1.0068x over previous
import jax
import jax.numpy as jnp
from jax.experimental import pallas as pl

_ROWS = 16384
_SEQ = 200
_DIM = 128
_BLK = 256


def _embed_kernel(pos_ref, w_ref, out_ref):
    posf = pos_ref[...].astype(jnp.float32)
    w0 = w_ref[0, :]
    diff = w_ref[1, :] - w0
    out_ref[...] = posf[:, :, None] * diff[None, None, :] + w0[None, None, :]


def kernel(pos, seg_embd_weight):
    pos = pos.astype(jnp.int32)
    grid = (_ROWS // _BLK,)
    return pl.pallas_call(
        _embed_kernel,
        grid=grid,
        in_specs=[
            pl.BlockSpec((_BLK, _SEQ), lambda i: (i, 0)),
            pl.BlockSpec((2, _DIM), lambda i: (0, 0)),
        ],
        out_specs=pl.BlockSpec((_BLK, _SEQ, _DIM), lambda i: (i, 0, 0)),
        out_shape=jax.ShapeDtypeStruct((_ROWS, _SEQ, _DIM), jnp.float32),
    )(pos, seg_embd_weight)
